# baseline (speedup 1.0000x reference)
import functools

import jax
import jax.numpy as jnp
from jax import lax
from jax.experimental import pallas as pl
from jax.experimental.pallas import tpu as pltpu
from jax.experimental.pallas import tpu_sc as plsc

NC, NS, L = 2, 16, 16
NW = NC * NS
BLK = 128
D = 4
CP = 128


@functools.partial(jax.jit, static_argnames=("nq", "h", "w"))
def _flosp_gather_sc(table, px, py, fov, *, nq, h, w):
    qpw = nq // NW
    nblk = qpw // BLK
    padrow = h * w
    assert nblk % D == 0

    mesh = plsc.VectorSubcoreMesh(core_axis_name="c", subcore_axis_name="s")

    def body(th, pxh, pyh, fovh, outh,
             px_v, py_v, fov_v, idx, buf,
             sg0, sg1, sg2, sg3, so0, so1, so2, so3):
        semg = (sg0, sg1, sg2, sg3)
        semo = (so0, so1, so2, so3)
        wid = lax.axis_index("s") * NC + lax.axis_index("c")
        qbase = wid * qpw
        pltpu.sync_copy(pxh.at[pl.ds(qbase, qpw)], px_v)
        pltpu.sync_copy(pyh.at[pl.ds(qbase, qpw)], py_v)
        pltpu.sync_copy(fovh.at[pl.ds(qbase, qpw)], fov_v)

        def idxpass(b, carry):
            for j in range(BLK // L):
                sl = pl.ds(b * BLK + j * L, L)
                iid = py_v[sl] * w + px_v[sl]
                iid = jnp.where(fov_v[sl] > 0, iid, padrow)
                idx[b, pl.ds(j * L, L)] = iid
            return carry

        lax.fori_loop(0, nblk, idxpass, 0)

        def gather_desc(b, s, fire):
            mk = pltpu.async_copy if fire else pltpu.make_async_copy
            return mk(th.at[idx.at[b]], buf.at[s], semg[s])

        def out_desc(b, s, fire):
            mk = pltpu.async_copy if fire else pltpu.make_async_copy
            return mk(buf.at[s], outh.at[pl.ds(qbase + b * BLK, BLK)], semo[s])

        for p in range(D - 1):
            gather_desc(p, p, fire=True)

        def step(bb, carry):
            for u in range(D):
                b = bb * D + u
                sf = (u + D - 1) % D
                @pl.when(b + D - 1 < nblk)
                def _():
                    @pl.when(b >= 1)
                    def _():
                        out_desc(b - 1, sf, fire=False).wait()
                    gather_desc(b + D - 1, sf, fire=True)
                gather_desc(b, u, fire=False).wait()
                out_desc(b, u, fire=True)
            return carry

        lax.fori_loop(0, nblk // D, step, 0)

        for u in range(D):
            out_desc(nblk - D + u, u, fire=False).wait()

    run = pl.kernel(
        body,
        out_type=jax.ShapeDtypeStruct((nq, CP), jnp.float32),
        mesh=mesh,
        scratch_types=[
            pltpu.VMEM((qpw,), jnp.int32),
            pltpu.VMEM((qpw,), jnp.int32),
            pltpu.VMEM((qpw,), jnp.int32),
            pltpu.VMEM((nblk, BLK), jnp.int32),
            pltpu.VMEM((D, BLK, CP), jnp.float32),
        ] + [pltpu.SemaphoreType.DMA] * (2 * D),
    )
    return run(table, px, py, fov)


def kernel(feat_s1, feat_s2, feat_s4, feat_s8, projected_pix, fov_mask):
    bs, num_cam, c, h, w = feat_s1.shape
    nq = projected_pix.shape[1]

    def up(f, k):
        a = f.reshape(c, h // k, w // k)
        return jnp.repeat(jnp.repeat(a, k, axis=1), k, axis=2)

    fused = ((feat_s1.reshape(c, h, w) + up(feat_s2, 2))
             + up(feat_s4, 4)) + up(feat_s8, 8)
    table = jnp.pad(fused.reshape(c, h * w).T, ((0, 8), (0, CP - c)))

    px = projected_pix[0, :, 0]
    py = projected_pix[0, :, 1]
    fov = fov_mask[0].astype(jnp.int32)

    y = _flosp_gather_sc(table, px, py, fov, nq=nq, h=h, w=w)
    return y[:, :c].T.reshape(bs, c, nq)


# speedup vs baseline: 1.9872x; 1.9872x over previous
import functools

import jax
import jax.numpy as jnp
from jax import lax
from jax.experimental import pallas as pl
from jax.experimental.pallas import tpu as pltpu
from jax.experimental.pallas import tpu_sc as plsc

NC, NS, L = 2, 16, 16
NW = NC * NS
BLK = 128
D = 4


@functools.partial(jax.jit, static_argnames=("nq", "c", "h", "w"))
def _flosp_gather_sc(table, px, py, fov, *, nq, c, h, w):
    qpw = nq // NW
    nblk = qpw // BLK
    padrow = h * w
    assert nblk % D == 0

    mesh = plsc.VectorSubcoreMesh(core_axis_name="c", subcore_axis_name="s")

    def body(th, pxh, pyh, fovh, outh,
             px_v, py_v, fov_v, idx, buf,
             sg0, sg1, sg2, sg3, so0, so1, so2, so3):
        semg = (sg0, sg1, sg2, sg3)
        semo = (so0, so1, so2, so3)
        wid = lax.axis_index("s") * NC + lax.axis_index("c")
        qbase = wid * qpw
        pltpu.sync_copy(pxh.at[pl.ds(qbase, qpw)], px_v)
        pltpu.sync_copy(pyh.at[pl.ds(qbase, qpw)], py_v)
        pltpu.sync_copy(fovh.at[pl.ds(qbase, qpw)], fov_v)

        def idxpass(b, carry):
            for j in range(BLK // L):
                sl = pl.ds(b * BLK + j * L, L)
                iid = py_v[sl] * w + px_v[sl]
                iid = jnp.where(fov_v[sl] > 0, iid, padrow)
                idx[b, pl.ds(j * L, L)] = iid
            return carry

        lax.fori_loop(0, nblk, idxpass, 0)

        def gather_desc(b, s, fire):
            mk = pltpu.async_copy if fire else pltpu.make_async_copy
            return mk(th.at[idx.at[b]], buf.at[s], semg[s])

        def out_desc(b, s, fire):
            mk = pltpu.async_copy if fire else pltpu.make_async_copy
            return mk(buf.at[s], outh.at[pl.ds(qbase + b * BLK, BLK)], semo[s])

        for p in range(D - 1):
            gather_desc(p, p, fire=True)

        def step(bb, carry):
            for u in range(D):
                b = bb * D + u
                sf = (u + D - 1) % D
                @pl.when(b + D - 1 < nblk)
                def _():
                    @pl.when(b >= 1)
                    def _():
                        out_desc(b - 1, sf, fire=False).wait()
                    gather_desc(b + D - 1, sf, fire=True)
                gather_desc(b, u, fire=False).wait()
                out_desc(b, u, fire=True)
            return carry

        lax.fori_loop(0, nblk // D, step, 0)

        for u in range(D):
            out_desc(nblk - D + u, u, fire=False).wait()

    run = pl.kernel(
        body,
        out_type=jax.ShapeDtypeStruct((nq, c), jnp.bfloat16),
        mesh=mesh,
        compiler_params=pltpu.CompilerParams(use_tc_tiling_on_sc=False),
        scratch_types=[
            pltpu.VMEM((qpw,), jnp.int32),
            pltpu.VMEM((qpw,), jnp.int32),
            pltpu.VMEM((qpw,), jnp.int32),
            pltpu.VMEM((nblk, BLK), jnp.int32),
            pltpu.VMEM((D, BLK, c), jnp.bfloat16),
        ] + [pltpu.SemaphoreType.DMA] * (2 * D),
    )
    return run(table, px, py, fov)


def kernel(feat_s1, feat_s2, feat_s4, feat_s8, projected_pix, fov_mask):
    bs, num_cam, c, h, w = feat_s1.shape
    nq = projected_pix.shape[1]

    def up(f, k):
        a = f.reshape(c, h // k, w // k)
        return jnp.repeat(jnp.repeat(a, k, axis=1), k, axis=2)

    fused = ((feat_s1.reshape(c, h, w) + up(feat_s2, 2))
             + up(feat_s4, 4)) + up(feat_s8, 8)
    table = fused.reshape(c, h * w).T.astype(jnp.bfloat16)
    table = jnp.concatenate(
        [table, jnp.zeros((1, c), jnp.bfloat16)], axis=0)

    px = projected_pix[0, :, 0]
    py = projected_pix[0, :, 1]
    fov = fov_mask[0].astype(jnp.int32)

    y = _flosp_gather_sc(table, px, py, fov, nq=nq, c=c, h=h, w=w)
    return y.T.astype(jnp.float32).reshape(bs, c, nq)


# speedup vs baseline: 2.1588x; 1.0863x over previous
import functools

import jax
import jax.numpy as jnp
from jax import lax
from jax.experimental import pallas as pl
from jax.experimental.pallas import tpu as pltpu
from jax.experimental.pallas import tpu_sc as plsc

NC, NS, L = 2, 16, 16
NW = NC * NS
BLK = 128
D = 4


@functools.partial(jax.jit, static_argnames=("nq", "c", "h", "w"))
def _flosp_gather_sc(table, px, py, fov, *, nq, c, h, w):
    qpw = nq // NW
    nblk = qpw // BLK
    padrow = h * w
    assert nblk % D == 0

    mesh = plsc.VectorSubcoreMesh(core_axis_name="c", subcore_axis_name="s")

    def body(th, pxh, pyh, fovh, outh,
             px_v, py_v, fov_v, idx, buf,
             sg0, sg1, sg2, sg3, so0, so1, so2, so3):
        semg = (sg0, sg1, sg2, sg3)
        semo = (so0, so1, so2, so3)
        wid = lax.axis_index("s") * NC + lax.axis_index("c")
        qbase = wid * qpw
        pltpu.sync_copy(pxh.at[pl.ds(qbase, qpw)], px_v)
        pltpu.sync_copy(pyh.at[pl.ds(qbase, qpw)], py_v)
        pltpu.sync_copy(fovh.at[pl.ds(qbase, qpw)], fov_v)

        def idxpass(b, carry):
            for j in range(BLK // L):
                sl = pl.ds(b * BLK + j * L, L)
                iid = py_v[sl] * w + px_v[sl]
                iid = jnp.where(fov_v[sl] > 0, iid, padrow)
                idx[b, pl.ds(j * L, L)] = iid
            return carry

        lax.fori_loop(0, nblk, idxpass, 0)

        def gather_desc(b, s, fire):
            mk = pltpu.async_copy if fire else pltpu.make_async_copy
            return mk(th.at[idx.at[b]], buf.at[s], semg[s])

        def out_desc(b, s, fire):
            mk = pltpu.async_copy if fire else pltpu.make_async_copy
            return mk(buf.at[s], outh.at[pl.ds(qbase + b * BLK, BLK)], semo[s])

        for p in range(D - 1):
            gather_desc(p, p, fire=True)

        def step(bb, carry):
            for u in range(D):
                b = bb * D + u
                sf = (u + D - 1) % D
                @pl.when(b + D - 1 < nblk)
                def _():
                    @pl.when(b >= 1)
                    def _():
                        out_desc(b - 1, sf, fire=False).wait()
                    gather_desc(b + D - 1, sf, fire=True)
                gather_desc(b, u, fire=False).wait()
                out_desc(b, u, fire=True)
            return carry

        lax.fori_loop(0, nblk // D, step, 0)

        for u in range(D):
            out_desc(nblk - D + u, u, fire=False).wait()

    run = pl.kernel(
        body,
        out_type=jax.ShapeDtypeStruct((nq, c), jnp.bfloat16),
        mesh=mesh,
        compiler_params=pltpu.CompilerParams(use_tc_tiling_on_sc=False),
        scratch_types=[
            pltpu.VMEM((qpw,), jnp.int32),
            pltpu.VMEM((qpw,), jnp.int32),
            pltpu.VMEM((qpw,), jnp.int32),
            pltpu.VMEM((nblk, BLK), jnp.int32),
            pltpu.VMEM((D, BLK, c), jnp.bfloat16),
        ] + [pltpu.SemaphoreType.DMA] * (2 * D),
    )
    return run(table, px, py, fov)


CB = 8


@functools.partial(jax.jit, static_argnames=("c", "h", "w"))
def _fuse_scales_tc(f1, f2, f4, f8, *, c, h, w):
    def up_block(a, k):
        hk, wk = h // k, w // k
        row = jax.lax.broadcasted_iota(jnp.int32, (wk, w), 0)
        col = jax.lax.broadcasted_iota(jnp.int32, (wk, w), 1)
        ux = (col // k == row).astype(jnp.float32)
        ax = jax.lax.dot_general(a, ux, (((2,), (0,)), ((), ())),
                                 preferred_element_type=jnp.float32)
        ay = jnp.broadcast_to(ax[:, :, None, :], (CB, hk, k, w))
        return ay.reshape(CB, h, w)

    def body(r1, r2, r4, r8, out):
        acc = ((r1[...] + up_block(r2[...], 2))
               + up_block(r4[...], 4)) + up_block(r8[...], 8)
        out[...] = acc.astype(jnp.bfloat16)

    def spec(k):
        return pl.BlockSpec((CB, h // k, w // k), lambda i: (i, 0, 0))

    return pl.pallas_call(
        body,
        grid=(c // CB,),
        in_specs=[spec(1), spec(2), spec(4), spec(8)],
        out_specs=spec(1),
        out_shape=jax.ShapeDtypeStruct((c, h, w), jnp.bfloat16),
    )(f1, f2, f4, f8)


def kernel(feat_s1, feat_s2, feat_s4, feat_s8, projected_pix, fov_mask):
    bs, num_cam, c, h, w = feat_s1.shape
    nq = projected_pix.shape[1]

    fused = _fuse_scales_tc(
        feat_s1.reshape(c, h, w), feat_s2.reshape(c, h // 2, w // 2),
        feat_s4.reshape(c, h // 4, w // 4), feat_s8.reshape(c, h // 8, w // 8),
        c=c, h=h, w=w)
    table = fused.reshape(c, h * w).T
    table = jnp.concatenate(
        [table, jnp.zeros((1, c), jnp.bfloat16)], axis=0)

    px = projected_pix[0, :, 0]
    py = projected_pix[0, :, 1]
    fov = fov_mask[0].astype(jnp.int32)

    y = _flosp_gather_sc(table, px, py, fov, nq=nq, c=c, h=h, w=w)
    return y.T.astype(jnp.float32).reshape(bs, c, nq)
